# baseline (speedup 1.0000x reference)
import functools

import jax
import jax.numpy as jnp
from jax import lax
from jax.experimental import pallas as pl
from jax.experimental.pallas import tpu as pltpu
from jax.experimental.pallas import tpu_sc as plsc

B = 4096
L = 200
E = 128
V = 1000

NC = 2
NS = 16
NW = NC * NS

CS = 32
NCS = E // CS
NBD = NW // NCS

ROWS = B * L
RPB = ROWS // NBD
R = 400
NGRP = RPB // R
NBLK = R // 16
TP = 33
NOBUF = 4
NIBUF = 3
IAHEAD = 2
PE_EXT = 224

_mesh = plsc.VectorSubcoreMesh(core_axis_name="c", subcore_axis_name="s")


@functools.partial(
    pl.kernel,
    out_type=jax.ShapeDtypeStruct((ROWS, E), jnp.float32),
    mesh=_mesh,
    compiler_params=pltpu.CompilerParams(use_tc_tiling_on_sc=False,
                                         needs_layout_passes=False),
    scratch_types=[
        pltpu.VMEM((V, TP), jnp.float32),
        pltpu.VMEM((NOBUF, R, TP), jnp.float32),
        pltpu.VMEM((NIBUF, R), jnp.int32),
        pltpu.VMEM((PE_EXT,), jnp.float32),
        pltpu.SemaphoreType.DMA((NIBUF,)),
        pltpu.SemaphoreType.DMA((NOBUF,)),
    ],
)
def _emb_kernel(table_hbm, xflat_hbm, out_hbm, table_v, obuf, ibuf, pe_v,
                isem, wsem):
    wid = lax.axis_index("s") * NC + lax.axis_index("c")
    cset = lax.rem(wid, NCS)
    band = wid // NCS
    c0 = cset * CS
    band_base = band * RPB

    pltpu.sync_copy(table_hbm.at[:, pl.ds(c0, CS)],
                    table_v.at[:, pl.ds(0, CS)])

    lanes = jnp.arange(16, dtype=jnp.int32)
    for k in range(PE_EXT // 16):
        p = lanes + (k * 16)
        pm = jnp.where(p >= L, p - L, p)
        val = jnp.where(pm % 2 == 0, pm.astype(jnp.float32) * jnp.float32(1e-8),
                        jnp.float32(1.0))
        pe_v[pl.ds(k * 16, 16)] = val

    def ifetch_desc(g, slot):
        return pltpu.make_async_copy(
            xflat_hbm.at[pl.ds(band_base + g * R, R)],
            ibuf.at[slot],
            isem.at[slot],
        )

    def write_desc(g, slot):
        return pltpu.make_async_copy(
            obuf.at[slot, :, pl.ds(0, CS)],
            out_hbm.at[pl.ds(band_base + g * R, R), pl.ds(c0, CS)],
            wsem.at[slot],
        )

    for g in range(IAHEAD):
        ifetch_desc(g, g % NIBUF).start()

    def group(g, carry):
        islot = lax.rem(g, NIBUF)
        oslot = lax.rem(g, NOBUF)

        @pl.when(g + IAHEAD < NGRP)
        def _():
            ifetch_desc(g + IAHEAD, lax.rem(g + IAHEAD, NIBUF)).start()

        ifetch_desc(g, islot).wait()

        @pl.when(g >= NOBUF)
        def _():
            write_desc(g - NOBUF, oslot).wait()

        out2d = obuf.at[oslot]
        for b in range(NBLK):
            rowv = ibuf[islot, pl.ds(b * 16, 16)]
            phase = (b * 16) % L
            padd = pe_v[pl.ds(phase, 16)]
            lrow = lanes + (b * 16)
            for c in range(CS):
                colv = jnp.full((16,), c, jnp.int32)
                gathered = plsc.load_gather(table_v, [rowv, colv])
                plsc.store_scatter(out2d, [lrow, colv], gathered + padd)

        write_desc(g, oslot).start()
        return carry

    lax.fori_loop(0, NGRP, group, 0)

    for g in range(NGRP - NOBUF, NGRP):
        write_desc(g, g % NOBUF).wait()


def kernel(x, input_table):
    x_flat = x.reshape(ROWS).astype(jnp.int32)
    out = _emb_kernel(input_table, x_flat)
    return out.reshape(B, L, E)


# speedup vs baseline: 10.3548x; 10.3548x over previous
import functools

import jax
import jax.numpy as jnp
from jax import lax
from jax.experimental import pallas as pl
from jax.experimental.pallas import tpu as pltpu
from jax.experimental.pallas import tpu_sc as plsc

B = 4096
L = 200
E = 128
V = 1000

NC = 2
NS = 16
NW = NC * NS

ROWS = B * L
RPW = ROWS // NW
G = L
NG = RPW // G
NBUF = 4
NCIDX = 3
GAHEAD = 2
VLO = 960
VHI = V - VLO
NBLK = 13

_mesh = plsc.VectorSubcoreMesh(core_axis_name="c", subcore_axis_name="s")


@functools.partial(
    pl.kernel,
    out_type=jax.ShapeDtypeStruct((ROWS, E), jnp.float32),
    mesh=_mesh,
    compiler_params=pltpu.CompilerParams(needs_layout_passes=False),
    scratch_types=[
        pltpu.VMEM_SHARED((VLO, E), jnp.float32),
        pltpu.VMEM((VHI, E), jnp.float32),
        pltpu.VMEM((NBUF, G, E), jnp.float32),
        pltpu.VMEM((NBUF * 208,), jnp.int32),
        pltpu.VMEM((NCIDX * 208,), jnp.int32),
        pltpu.SemaphoreType.DMA((NBUF,)),
        pltpu.SemaphoreType.DMA((NBUF,)),
        pltpu.SemaphoreType.DMA((NBUF,)),
    ],
)
def _emb_kernel(table_hbm, xflat_hbm, out_hbm, table_sp, high_v, bufs,
                iring, cidx, isem, gsem, wsem):
    wid = lax.axis_index("s") * NC + lax.axis_index("c")
    base = wid * RPW
    lanes = jnp.arange(16, dtype=jnp.int32)

    pltpu.sync_copy(table_hbm.at[pl.ds(0, VLO)], table_sp)
    pltpu.sync_copy(table_hbm.at[pl.ds(VLO, VHI)], high_v)

    def ifetch_desc(g, slot):
        return pltpu.make_async_copy(
            xflat_hbm.at[pl.ds(base + g * G, G)],
            iring.at[pl.ds(slot * 208, G)],
            isem.at[slot],
        )

    def clamp_group(g):
        s4 = lax.rem(g, NBUF)
        s3 = lax.rem(g, NCIDX)
        for k in range(NBLK):
            v = iring[pl.ds(s4 * 208 + k * 16, 16)]
            cidx[pl.ds(s3 * 208 + k * 16, 16)] = jnp.minimum(
                v, jnp.int32(VLO - 1))

    def gather_descs(g):
        slot = lax.rem(g, NBUF)
        s3 = lax.rem(g, NCIDX)
        d1 = pltpu.make_async_copy(
            table_sp.at[cidx.at[pl.ds(s3 * 208, 128)]],
            bufs.at[slot, pl.ds(0, 128)],
            gsem.at[slot],
        )
        d2 = pltpu.make_async_copy(
            table_sp.at[cidx.at[pl.ds(s3 * 208 + 128, G - 128)]],
            bufs.at[slot, pl.ds(128, G - 128)],
            gsem.at[slot],
        )
        return d1, d2

    def write_desc(g, slot):
        return pltpu.make_async_copy(
            bufs.at[slot],
            out_hbm.at[pl.ds(base + g * G, G)],
            wsem.at[slot],
        )

    for g in range(3):
        ifetch_desc(g, g).start()
    for g in range(GAHEAD):
        ifetch_desc(g, g).wait()
        clamp_group(g)
        d1, d2 = gather_descs(g)
        d1.start()
        d2.start()

    def group(g, carry):
        slot = lax.rem(g, NBUF)

        @pl.when(g + 3 < NG)
        def _():
            ifetch_desc(g + 3, lax.rem(g + 3, NBUF)).start()

        @pl.when(g + GAHEAD < NG)
        def _():
            g2 = g + GAHEAD
            ifetch_desc(g2, lax.rem(g2, NBUF)).wait()
            clamp_group(g2)

            @pl.when(g2 >= NBUF)
            def _():
                write_desc(g2 - NBUF, lax.rem(g2, NBUF)).wait()

            d1, d2 = gather_descs(g2)
            d1.start()
            d2.start()

        d1, d2 = gather_descs(g)
        d1.wait()
        d2.wait()

        for blk in range(NBLK):
            off = blk * 16
            idxo = iring[pl.ds(slot * 208 + off, 16)]
            valid = idxo >= jnp.int32(VLO)
            if off + 16 > G:
                valid = valid & (lanes < (G - off))

            @pl.when(jnp.any(valid))
            def _(off=off, idxo=idxo, valid=valid):
                def fix_one(m):
                    lane = lax.reduce_max(
                        jnp.where(m, lanes, jnp.int32(-1)), (0,))
                    hv = lax.reduce_max(
                        jnp.where(lanes == lane, idxo, jnp.int32(-1)), (0,))
                    row = off + lane
                    hrow = hv - jnp.int32(VLO)
                    for e in range(E // 16):
                        bufs[slot, row, pl.ds(e * 16, 16)] = (
                            high_v[hrow, pl.ds(e * 16, 16)])
                    return m & (lanes != lane)

                lax.while_loop(lambda m: jnp.any(m), fix_one, valid)

        def addrow(j, carry2):
            jf = j.astype(jnp.float32)
            val = jnp.where(j % 2 == 0, jf * jnp.float32(1e-8),
                            jnp.float32(1.0))
            for e in range(E // 16):
                bufs[slot, j, pl.ds(e * 16, 16)] = (
                    bufs[slot, j, pl.ds(e * 16, 16)] + val)
            return carry2

        lax.fori_loop(0, G, addrow, 0, unroll=2)

        write_desc(g, slot).start()
        return carry

    lax.fori_loop(0, NG, group, 0)

    for g in range(NG - NBUF, NG):
        write_desc(g, g % NBUF).wait()


def kernel(x, input_table):
    x_flat = x.reshape(ROWS).astype(jnp.int32)
    out = _emb_kernel(input_table, x_flat)
    return out.reshape(B, L, E)
